# baseline (speedup 1.0000x reference)
import jax
import jax.numpy as jnp
from jax.experimental import pallas as pl
from jax.experimental.pallas import tpu as pltpu

N_EDGES = 320000
N_NODES = 10000
D = 128

CHUNK = 4000
NCHUNK = N_EDGES // CHUNK
RING = 8
NSTEP = 10
CPS = NCHUNK // NSTEP
NODE_BLOCK = N_NODES // NSTEP
ACC_ROWS = 200


def _body(neigh_hbm, src_ref, w_self_ref, w_aggr_ref, out_hbm,
          nb0, nb1, nb2, nb3, nb4, nb5, nb6, nb7,
          acc_ref, ob0, ob1, bcast_ref,
          ns0, ns1, ns2, ns3, ns4, ns5, ns6, ns7, os0, os1, bsem):
    i = pl.program_id(0)
    nbufs = (nb0, nb1, nb2, nb3, nb4, nb5, nb6, nb7)
    nsems = (ns0, ns1, ns2, ns3, ns4, ns5, ns6, ns7)

    @pl.when(i == 0)
    def _():
        acc_ref[...] = jnp.zeros_like(acc_ref)
        for s in range(RING):
            pltpu.async_copy(neigh_hbm.at[pl.ds(s * CHUNK, CHUNK)],
                             nbufs[s], nsems[s])

    for k in range(CPS):
        slot = k % RING
        c = i * CPS + k
        pltpu.make_async_copy(neigh_hbm.at[pl.ds(0, CHUNK)],
                              nbufs[slot], nsems[slot]).wait()
        x = nbufs[slot][...]
        acc_ref[...] += jnp.sum(
            x.reshape(CHUNK // ACC_ROWS, ACC_ROWS, D), axis=0)

        @pl.when(c + RING < NCHUNK)
        def _():
            pltpu.async_copy(
                neigh_hbm.at[pl.ds((c + RING) * CHUNK, CHUNK)],
                nbufs[slot], nsems[slot])

    @pl.when(i >= 2)
    def _():
        @pl.when(i % 2 == 0)
        def _():
            pltpu.make_async_copy(ob0, out_hbm.at[pl.ds(0, NODE_BLOCK),
                                                  pl.ds(0, D)], os0).wait()

        @pl.when(i % 2 == 1)
        def _():
            pltpu.make_async_copy(ob1, out_hbm.at[pl.ds(0, NODE_BLOCK),
                                                  pl.ds(0, D)], os1).wait()

    self_hidden = jnp.maximum(
        jnp.dot(src_ref[...], w_self_ref[...],
                preferred_element_type=jnp.float32), 0.0)

    @pl.when(i % 2 == 0)
    def _():
        ob0[...] = self_hidden
        pltpu.async_copy(
            ob0, out_hbm.at[pl.ds(i * NODE_BLOCK, NODE_BLOCK), pl.ds(0, D)],
            os0)

    @pl.when(i % 2 == 1)
    def _():
        ob1[...] = self_hidden
        pltpu.async_copy(
            ob1, out_hbm.at[pl.ds(i * NODE_BLOCK, NODE_BLOCK), pl.ds(0, D)],
            os1)

    @pl.when(i == NSTEP - 1)
    def _():
        total = jnp.sum(acc_ref[...], axis=0, keepdims=True)
        mean = total * (1.0 / N_EDGES)
        nh = jnp.maximum(
            jnp.dot(mean, w_aggr_ref[...],
                    preferred_element_type=jnp.float32), 0.0)
        bcast_ref[...] = jnp.broadcast_to(nh, (NODE_BLOCK, D))
        for b in range(NSTEP):
            pltpu.async_copy(
                bcast_ref,
                out_hbm.at[pl.ds(b * NODE_BLOCK, NODE_BLOCK), pl.ds(D, D)],
                bsem)
        for b in range(NSTEP):
            pltpu.make_async_copy(
                bcast_ref,
                out_hbm.at[pl.ds(b * NODE_BLOCK, NODE_BLOCK), pl.ds(D, D)],
                bsem).wait()
        pltpu.make_async_copy(ob0, out_hbm.at[pl.ds(0, NODE_BLOCK),
                                              pl.ds(0, D)], os0).wait()
        pltpu.make_async_copy(ob1, out_hbm.at[pl.ds(0, NODE_BLOCK),
                                              pl.ds(0, D)], os1).wait()


def kernel(src_node_features, neighbor_node_features, W_aggr, W_self):
    out = pl.pallas_call(
        _body,
        grid=(NSTEP,),
        in_specs=[
            pl.BlockSpec(memory_space=pl.ANY),
            pl.BlockSpec((NODE_BLOCK, D), lambda i: (i, 0)),
            pl.BlockSpec((D, D), lambda i: (0, 0)),
            pl.BlockSpec((D, D), lambda i: (0, 0)),
        ],
        out_specs=pl.BlockSpec(memory_space=pl.ANY),
        out_shape=jax.ShapeDtypeStruct((N_NODES, 2 * D), jnp.float32),
        scratch_shapes=[
            pltpu.VMEM((CHUNK, D), jnp.float32),
            pltpu.VMEM((CHUNK, D), jnp.float32),
            pltpu.VMEM((CHUNK, D), jnp.float32),
            pltpu.VMEM((CHUNK, D), jnp.float32),
            pltpu.VMEM((CHUNK, D), jnp.float32),
            pltpu.VMEM((CHUNK, D), jnp.float32),
            pltpu.VMEM((CHUNK, D), jnp.float32),
            pltpu.VMEM((CHUNK, D), jnp.float32),
            pltpu.VMEM((ACC_ROWS, D), jnp.float32),
            pltpu.VMEM((NODE_BLOCK, D), jnp.float32),
            pltpu.VMEM((NODE_BLOCK, D), jnp.float32),
            pltpu.VMEM((NODE_BLOCK, D), jnp.float32),
            pltpu.SemaphoreType.DMA,
            pltpu.SemaphoreType.DMA,
            pltpu.SemaphoreType.DMA,
            pltpu.SemaphoreType.DMA,
            pltpu.SemaphoreType.DMA,
            pltpu.SemaphoreType.DMA,
            pltpu.SemaphoreType.DMA,
            pltpu.SemaphoreType.DMA,
            pltpu.SemaphoreType.DMA,
            pltpu.SemaphoreType.DMA,
            pltpu.SemaphoreType.DMA,
        ],
    )(neighbor_node_features, src_node_features, W_self, W_aggr)
    return out


# speedup vs baseline: 1.0102x; 1.0102x over previous
import jax
import jax.numpy as jnp
from jax.experimental import pallas as pl
from jax.experimental.pallas import tpu as pltpu

N_EDGES = 320000
N_NODES = 10000
D = 128

CHUNK = 8000
NCHUNK = N_EDGES // CHUNK
RING = 4
NSTEP = 5
CPS = NCHUNK // NSTEP
NODE_BLOCK = N_NODES // NSTEP
ACC_ROWS = 200


def _body(neigh_hbm, src_ref, w_self_ref, w_aggr_ref, out_hbm,
          nb0, nb1, nb2, nb3, acc_ref, ob0, ob1, bcast_ref,
          ns0, ns1, ns2, ns3, os0, os1, bsem):
    i = pl.program_id(0)
    nbufs = (nb0, nb1, nb2, nb3)
    nsems = (ns0, ns1, ns2, ns3)

    @pl.when(i == 0)
    def _():
        acc_ref[...] = jnp.zeros_like(acc_ref)
        for s in range(RING):
            pltpu.async_copy(neigh_hbm.at[pl.ds(s * CHUNK, CHUNK)],
                             nbufs[s], nsems[s])

    for k in range(CPS):
        slot = k % RING
        c = i * CPS + k
        pltpu.make_async_copy(neigh_hbm.at[pl.ds(0, CHUNK)],
                              nbufs[slot], nsems[slot]).wait()
        x = nbufs[slot][...]
        acc_ref[...] += jnp.sum(
            x.reshape(CHUNK // ACC_ROWS, ACC_ROWS, D), axis=0)

        @pl.when(c + RING < NCHUNK)
        def _():
            pltpu.async_copy(
                neigh_hbm.at[pl.ds((c + RING) * CHUNK, CHUNK)],
                nbufs[slot], nsems[slot])

    @pl.when(i >= 2)
    def _():
        @pl.when(i % 2 == 0)
        def _():
            pltpu.make_async_copy(ob0, out_hbm.at[pl.ds(0, NODE_BLOCK),
                                                  pl.ds(0, D)], os0).wait()

        @pl.when(i % 2 == 1)
        def _():
            pltpu.make_async_copy(ob1, out_hbm.at[pl.ds(0, NODE_BLOCK),
                                                  pl.ds(0, D)], os1).wait()

    self_hidden = jnp.maximum(
        jnp.dot(src_ref[...], w_self_ref[...],
                preferred_element_type=jnp.float32), 0.0)

    @pl.when(i % 2 == 0)
    def _():
        ob0[...] = self_hidden
        pltpu.async_copy(
            ob0, out_hbm.at[pl.ds(i * NODE_BLOCK, NODE_BLOCK), pl.ds(0, D)],
            os0)

    @pl.when(i % 2 == 1)
    def _():
        ob1[...] = self_hidden
        pltpu.async_copy(
            ob1, out_hbm.at[pl.ds(i * NODE_BLOCK, NODE_BLOCK), pl.ds(0, D)],
            os1)

    @pl.when(i == NSTEP - 1)
    def _():
        total = jnp.sum(acc_ref[...], axis=0, keepdims=True)
        mean = total * (1.0 / N_EDGES)
        nh = jnp.maximum(
            jnp.dot(mean, w_aggr_ref[...],
                    preferred_element_type=jnp.float32), 0.0)
        bcast_ref[...] = jnp.broadcast_to(nh, (NODE_BLOCK, D))
        for b in range(NSTEP):
            pltpu.async_copy(
                bcast_ref,
                out_hbm.at[pl.ds(b * NODE_BLOCK, NODE_BLOCK), pl.ds(D, D)],
                bsem)
        for b in range(NSTEP):
            pltpu.make_async_copy(
                bcast_ref,
                out_hbm.at[pl.ds(b * NODE_BLOCK, NODE_BLOCK), pl.ds(D, D)],
                bsem).wait()
        pltpu.make_async_copy(ob0, out_hbm.at[pl.ds(0, NODE_BLOCK),
                                              pl.ds(0, D)], os0).wait()
        pltpu.make_async_copy(ob1, out_hbm.at[pl.ds(0, NODE_BLOCK),
                                              pl.ds(0, D)], os1).wait()


def kernel(src_node_features, neighbor_node_features, W_aggr, W_self):
    out = pl.pallas_call(
        _body,
        grid=(NSTEP,),
        in_specs=[
            pl.BlockSpec(memory_space=pl.ANY),
            pl.BlockSpec((NODE_BLOCK, D), lambda i: (i, 0)),
            pl.BlockSpec((D, D), lambda i: (0, 0)),
            pl.BlockSpec((D, D), lambda i: (0, 0)),
        ],
        out_specs=pl.BlockSpec(memory_space=pl.ANY),
        out_shape=jax.ShapeDtypeStruct((N_NODES, 2 * D), jnp.float32),
        scratch_shapes=[
            pltpu.VMEM((CHUNK, D), jnp.float32),
            pltpu.VMEM((CHUNK, D), jnp.float32),
            pltpu.VMEM((CHUNK, D), jnp.float32),
            pltpu.VMEM((CHUNK, D), jnp.float32),
            pltpu.VMEM((ACC_ROWS, D), jnp.float32),
            pltpu.VMEM((NODE_BLOCK, D), jnp.float32),
            pltpu.VMEM((NODE_BLOCK, D), jnp.float32),
            pltpu.VMEM((NODE_BLOCK, D), jnp.float32),
            pltpu.SemaphoreType.DMA,
            pltpu.SemaphoreType.DMA,
            pltpu.SemaphoreType.DMA,
            pltpu.SemaphoreType.DMA,
            pltpu.SemaphoreType.DMA,
            pltpu.SemaphoreType.DMA,
            pltpu.SemaphoreType.DMA,
        ],
    )(neighbor_node_features, src_node_features, W_self, W_aggr)
    return out
